# baseline (speedup 1.0000x reference)
import jax
import jax.numpy as jnp
from jax import lax
from jax.experimental import pallas as pl
from jax.experimental.pallas import tpu as pltpu
from jax.experimental.pallas import tpu_sc as plsc
import functools

N = 10000
D = 256
NV = D // 16
S = 64
CH = 80
NCHUNK = N // CH
NC = 2
NS = 16
NW = NC * NS
CPW = 4


def _sc_pool_body(x_hbm, b2d_hbm, psum_hbm, idx_v, x_v, acc_v, sems):
    core = lax.axis_index("c")
    sid = lax.axis_index("s")
    wid = sid * NC + core

    for j in range(CPW):
        c = wid + NW * j

        @pl.when(c < NCHUNK)
        def _(c=c, j=j):
            pltpu.async_copy(b2d_hbm.at[c], idx_v.at[j], sems.at[j])
            pltpu.async_copy(x_hbm.at[pl.ds(c * CH, CH)], x_v.at[j],
                             sems.at[CPW + j])

    zeros16 = jnp.zeros((16,), jnp.float32)

    def zero_body(r, carry):
        for j in range(NV):
            acc_v[r, pl.ds(j * 16, 16)] = zeros16
        return carry

    lax.fori_loop(0, S, zero_body, jnp.int32(0))

    def chunk_body(jj, carry):
        c = wid + NW * jj

        @pl.when(c < NCHUNK)
        def _():
            pltpu.make_async_copy(b2d_hbm.at[c], idx_v.at[jj],
                                  sems.at[jj]).wait()
            pltpu.make_async_copy(x_hbm.at[pl.ds(c * CH, CH)], x_v.at[jj],
                                  sems.at[CPW + jj]).wait()

            for g in range(CH // 16):
                idx16 = idx_v[jj, pl.ds(g * 16, 16)]
                for l in range(16):
                    s = idx16[l]
                    r = g * 16 + l
                    vals = [x_v[jj, r, pl.ds(j * 16, 16)]
                            for j in range(NV)]
                    for j in range(NV):
                        plsc.addupdate(acc_v.at[s, pl.ds(j * 16, 16)],
                                       vals[j])

        return carry

    lax.fori_loop(0, CPW, chunk_body, jnp.int32(0))

    pltpu.sync_copy(acc_v, psum_hbm.at[wid])


_sc_pool = functools.partial(
    pl.kernel,
    out_type=[
        jax.ShapeDtypeStruct((NW, S, D), jnp.float32),
    ],
    mesh=plsc.VectorSubcoreMesh(core_axis_name="c", subcore_axis_name="s"),
    scratch_types=[
        pltpu.VMEM((CPW, CH), jnp.int32),
        pltpu.VMEM((CPW, CH, D), jnp.float32),
        pltpu.VMEM((S, D), jnp.float32),
        pltpu.SemaphoreType.DMA((2 * CPW,)),
    ],
)(_sc_pool_body)


def _combine_body(ps_ref, b_ref, o_ref):
    sums = jnp.sum(ps_ref[...], axis=0)
    seg = lax.broadcasted_iota(jnp.int32, (S, N), 0)
    onehot = (b_ref[...] == seg).astype(jnp.float32)
    cnt = jnp.sum(onehot, axis=1, keepdims=True)
    o_ref[...] = sums / jnp.maximum(cnt, 1.0)


_combine = pl.pallas_call(
    _combine_body,
    out_shape=jax.ShapeDtypeStruct((S, D), jnp.float32),
)


@jax.jit
def kernel(x, edge_index, batch):
    del edge_index
    b32 = batch.astype(jnp.int32)
    (psum,) = _sc_pool(x, b32.reshape(NCHUNK, CH))
    return _combine(psum, b32.reshape(1, N))


# speedup vs baseline: 1.2628x; 1.2628x over previous
import jax
import jax.numpy as jnp
from jax import lax
from jax.experimental import pallas as pl
from jax.experimental.pallas import tpu as pltpu
from jax.experimental.pallas import tpu_sc as plsc
import functools

N = 10000
D = 256
NV = D // 16
S = 64
CH = 80
NCHUNK = N // CH
NC = 2
NS = 16
NW = NC * NS
CPW = 4


def _sc_pool_body(x_hbm, b2d_hbm, zeros_hbm, psum_hbm,
                  idx_v, x_v, acc_v, sems):
    core = lax.axis_index("c")
    sid = lax.axis_index("s")
    wid = sid * NC + core

    zcopy = pltpu.async_copy(zeros_hbm, acc_v, sems.at[2 * CPW])
    for j in range(CPW):
        c = wid + NW * j

        @pl.when(c < NCHUNK)
        def _(c=c, j=j):
            pltpu.async_copy(b2d_hbm.at[c], idx_v.at[j], sems.at[j])
            pltpu.async_copy(x_hbm.at[pl.ds(c * CH, CH)], x_v.at[j],
                             sems.at[CPW + j])

    zcopy.wait()

    for j in range(CPW):
        c = wid + NW * j

        @pl.when(c < NCHUNK)
        def _(c=c, j=j):
            pltpu.make_async_copy(b2d_hbm.at[c], idx_v.at[j],
                                  sems.at[j]).wait()
            pltpu.make_async_copy(x_hbm.at[pl.ds(c * CH, CH)], x_v.at[j],
                                  sems.at[CPW + j]).wait()

            def group_body(g, carry):
                idx16 = idx_v[j, pl.ds(g * 16, 16)]
                for l in range(16):
                    s = idx16[l]
                    r = g * 16 + l
                    vals = [x_v[j, r, pl.ds(k * 16, 16)]
                            for k in range(NV)]
                    for k in range(NV):
                        plsc.addupdate(acc_v.at[s, pl.ds(k * 16, 16)],
                                       vals[k])
                return carry

            lax.fori_loop(0, CH // 16, group_body, jnp.int32(0))

    pltpu.sync_copy(acc_v, psum_hbm.at[wid])


_sc_pool = functools.partial(
    pl.kernel,
    out_type=[
        jax.ShapeDtypeStruct((NW, S, D), jnp.float32),
    ],
    mesh=plsc.VectorSubcoreMesh(core_axis_name="c", subcore_axis_name="s"),
    scratch_types=[
        pltpu.VMEM((CPW, CH), jnp.int32),
        pltpu.VMEM((CPW, CH, D), jnp.float32),
        pltpu.VMEM((S, D), jnp.float32),
        pltpu.SemaphoreType.DMA((2 * CPW + 1,)),
    ],
)(_sc_pool_body)


def _combine_body(ps_ref, b_ref, o_ref):
    sums = jnp.sum(ps_ref[...], axis=0)
    seg = lax.broadcasted_iota(jnp.int32, (S, N), 0)
    onehot = (b_ref[...] == seg).astype(jnp.float32)
    cnt = jnp.sum(onehot, axis=1, keepdims=True)
    o_ref[...] = sums / jnp.maximum(cnt, 1.0)


_combine = pl.pallas_call(
    _combine_body,
    out_shape=jax.ShapeDtypeStruct((S, D), jnp.float32),
)


@jax.jit
def kernel(x, edge_index, batch):
    del edge_index
    b32 = batch.astype(jnp.int32)
    zeros = jnp.zeros((S, D), jnp.float32)
    (psum,) = _sc_pool(x, b32.reshape(NCHUNK, CH), zeros)
    return _combine(psum, b32.reshape(1, N))
